# baseline (speedup 1.0000x reference)
import jax
import jax.numpy as jnp
from jax.experimental import pallas as pl

_ETA = 0.05
_D = 16
_IT_OLD = 4
_IT_DELTA = 4
_IT_FINAL = 7
_BB = 512


def _renorm(v):
    return v * jax.lax.rsqrt(jnp.sum(v * v, axis=(0, 1), keepdims=True) + 1e-30)


def _skew_spec_norm(S, iters):
    fro2 = jnp.sum(S * S, axis=(0, 1), keepdims=True)
    Ss = S * jax.lax.rsqrt(fro2 + 1e-30)
    v = jnp.ones((1, _D, S.shape[2]), dtype=S.dtype)
    half = iters // 2
    for i in range(half):
        w = jnp.sum(Ss * v, axis=1, keepdims=True)
        v = jnp.sum(Ss * w, axis=0, keepdims=True)
        if i == half // 2:
            v = _renorm(v)
    u = jnp.sum(Ss * v, axis=1, keepdims=True)
    num = jnp.sum(u * u, axis=(0, 1), keepdims=True)
    den = jnp.sum(v * v, axis=(0, 1), keepdims=True) + 1e-38
    return jnp.sqrt(fro2 * num / den)


def _skew_spec_norm_sq(S, doubles):
    fro2 = jnp.sum(S * S, axis=(0, 1), keepdims=True)
    Ss = S * jax.lax.rsqrt(fro2 + 1e-30)
    M = _matmul(Ss, Ss)
    v = jnp.ones((1, _D, S.shape[2]), dtype=S.dtype)
    for i in range(doubles):
        w = jnp.sum(M * v, axis=1, keepdims=True)
        v = jnp.sum(M * w, axis=0, keepdims=True)
        if i == doubles // 2:
            v = _renorm(v)
    u = jnp.sum(M * v, axis=1, keepdims=True)
    num = jnp.sum(u * u, axis=(0, 1), keepdims=True)
    den = jnp.sum(v * v, axis=(0, 1), keepdims=True) + 1e-38
    lam = jnp.sqrt(num / den)
    return jnp.sqrt(fro2 * lam)


def _gen_spec_norm(A, iters):
    fro2 = jnp.sum(A * A, axis=(0, 1), keepdims=True)
    As = A * jax.lax.rsqrt(fro2 + 1e-30)
    v = jnp.ones((1, _D, A.shape[2]), dtype=A.dtype)
    for i in range(iters):
        u = jnp.sum(As * v, axis=1, keepdims=True)
        v = jnp.sum(As * u, axis=0, keepdims=True)
        if i == iters // 2:
            v = _renorm(v)
    u = jnp.sum(As * v, axis=1, keepdims=True)
    num = jnp.sum(u * u, axis=(0, 1), keepdims=True)
    den = jnp.sum(v * v, axis=(0, 1), keepdims=True) + 1e-38
    return jnp.sqrt(fro2 * num / den)


def _matmul(X, Y):
    acc = X[:, 0:1, :] * Y[0:1, :, :]
    for k in range(1, _D):
        acc = acc + X[:, k:k + 1, :] * Y[k:k + 1, :, :]
    return acc


def _body(a_ref, d_ref, o_ref):
    A = a_ref[...]
    Dep = d_ref[...]
    K = 0.5 * (A - jnp.swapaxes(A, 0, 1))
    dA = 0.5 * (Dep - jnp.swapaxes(Dep, 0, 1))

    norm_old = _gen_spec_norm(A, _IT_OLD)
    norm_delta = _ETA * _skew_spec_norm(dA, _IT_DELTA)
    avail = jnp.clip(0.693 - norm_old, 1e-8, None)
    sc = jnp.minimum(avail / (norm_delta + 1e-8), 1.0)

    P = _matmul(K, dA)
    Knew = K + (_ETA * sc) * dA + (0.5 * _ETA * sc) * (P - jnp.swapaxes(P, 0, 1))

    fn = _skew_spec_norm_sq(Knew, _IT_FINAL)
    lim = jnp.minimum(0.693 / (fn + 1e-8), 1.0)
    o_ref[...] = Knew * lim


def _run(A3, D3, interpret=False):
    B = A3.shape[2]
    grid = (B // _BB,)
    spec = pl.BlockSpec((_D, _D, _BB), lambda j: (0, 0, j))
    return pl.pallas_call(
        _body,
        grid=grid,
        in_specs=[spec, spec],
        out_specs=spec,
        out_shape=jax.ShapeDtypeStruct((_D, _D, B), jnp.float32),
        interpret=interpret,
    )(A3, D3)


def kernel(A_old, delta_A_episode):
    B = A_old.shape[0]
    A3 = A_old.reshape(B, _D * _D).T.reshape(_D, _D, B)
    D3 = delta_A_episode.reshape(B, _D * _D).T.reshape(_D, _D, B)
    out3 = _run(A3, D3)
    return out3.reshape(_D * _D, B).T.reshape(B, _D, _D)


# speedup vs baseline: 1.5300x; 1.5300x over previous
import jax
import jax.numpy as jnp
from jax.experimental import pallas as pl

_ETA = 0.05
_D = 16
_IT_OLD = 3
_IT_DELTA = 4
_IT_FINAL = 6
_BB = 512


def _renorm(v):
    return v * jax.lax.rsqrt(jnp.sum(v * v, axis=(0, 1), keepdims=True) + 1e-30)


def _skew_spec_norm(S, iters):
    fro2 = jnp.sum(S * S, axis=(0, 1), keepdims=True)
    Ss = S * jax.lax.rsqrt(fro2 + 1e-30)
    v = jnp.ones((1, _D, S.shape[2]), dtype=S.dtype)
    half = iters // 2
    for i in range(half):
        w = jnp.sum(Ss * v, axis=1, keepdims=True)
        v = jnp.sum(Ss * w, axis=0, keepdims=True)
        if i == half // 2:
            v = _renorm(v)
    u = jnp.sum(Ss * v, axis=1, keepdims=True)
    num = jnp.sum(u * u, axis=(0, 1), keepdims=True)
    den = jnp.sum(v * v, axis=(0, 1), keepdims=True) + 1e-38
    return jnp.sqrt(fro2 * num / den)


def _skew_spec_norm_sq(S, doubles):
    fro2 = jnp.sum(S * S, axis=(0, 1), keepdims=True)
    Ss = S * jax.lax.rsqrt(fro2 + 1e-30)
    M = _matmul(Ss, Ss)
    v = jnp.ones((1, _D, S.shape[2]), dtype=S.dtype)
    for i in range(doubles):
        w = jnp.sum(M * v, axis=1, keepdims=True)
        v = jnp.sum(M * w, axis=0, keepdims=True)
        if i == doubles // 2:
            v = _renorm(v)
    u = jnp.sum(M * v, axis=1, keepdims=True)
    num = jnp.sum(u * u, axis=(0, 1), keepdims=True)
    den = jnp.sum(v * v, axis=(0, 1), keepdims=True) + 1e-38
    lam = jnp.sqrt(num / den)
    return jnp.sqrt(fro2 * lam)


def _gen_spec_norm(A, iters):
    fro2 = jnp.sum(A * A, axis=(0, 1), keepdims=True)
    As = A * jax.lax.rsqrt(fro2 + 1e-30)
    v = jnp.ones((1, _D, A.shape[2]), dtype=A.dtype)
    for i in range(iters):
        u = jnp.sum(As * v, axis=1, keepdims=True)
        v = jnp.sum(As * u, axis=0, keepdims=True)
        if i == iters // 2:
            v = _renorm(v)
    u = jnp.sum(As * v, axis=1, keepdims=True)
    num = jnp.sum(u * u, axis=(0, 1), keepdims=True)
    den = jnp.sum(v * v, axis=(0, 1), keepdims=True) + 1e-38
    return jnp.sqrt(fro2 * num / den)


def _matmul(X, Y):
    acc = X[:, 0:1, :] * Y[0:1, :, :]
    for k in range(1, _D):
        acc = acc + X[:, k:k + 1, :] * Y[k:k + 1, :, :]
    return acc


def _body(a_ref, d_ref, o_ref):
    A = a_ref[...]
    Dep = d_ref[...]
    K = 0.5 * (A - jnp.swapaxes(A, 0, 1))
    dA = 0.5 * (Dep - jnp.swapaxes(Dep, 0, 1))

    norm_old = _gen_spec_norm(A, _IT_OLD)
    norm_delta = _ETA * _skew_spec_norm(dA, _IT_DELTA)
    avail = jnp.clip(0.693 - norm_old, 1e-8, None)
    sc = jnp.minimum(avail / (norm_delta + 1e-8), 1.0)

    P = _matmul(K, dA)
    Knew = K + (_ETA * sc) * dA + (0.5 * _ETA * sc) * (P - jnp.swapaxes(P, 0, 1))

    fn = _skew_spec_norm_sq(Knew, _IT_FINAL)
    lim = jnp.minimum(0.693 / (fn + 1e-8), 1.0)
    o_ref[...] = Knew * lim


def _run(A3, D3, interpret=False):
    B = A3.shape[2]
    grid = (B // _BB,)
    spec = pl.BlockSpec((_D, _D, _BB), lambda j: (0, 0, j))
    return pl.pallas_call(
        _body,
        grid=grid,
        in_specs=[spec, spec],
        out_specs=spec,
        out_shape=jax.ShapeDtypeStruct((_D, _D, B), jnp.float32),
        interpret=interpret,
    )(A3, D3)


def kernel(A_old, delta_A_episode):
    B = A_old.shape[0]
    A3 = A_old.reshape(B, _D * _D).T.reshape(_D, _D, B)
    D3 = delta_A_episode.reshape(B, _D * _D).T.reshape(_D, _D, B)
    out3 = _run(A3, D3)
    return out3.reshape(_D * _D, B).T.reshape(B, _D, _D)
